# baseline (speedup 1.0000x reference)
import dataclasses
import jax
import jax.numpy as jnp
from jax import lax
from jax.experimental import pallas as pl
from jax.experimental.pallas import tpu as pltpu
from jax.experimental.pallas import tpu_sc as plsc

SEQ = 2048
DIM = 4096
NW = 32
ROWS_PER_W = SEQ // NW
CH = 8
NCH = ROWS_PER_W // CH


def _sc_body(m_hbm, a_hbm, x_hbm, o_hbm,
             m_vmem, buf0, buf1, si0, si1, so0, so1, sm):
    wid = lax.axis_index("s") * 2 + lax.axis_index("c")
    base = wid * ROWS_PER_W
    pltpu.async_copy(m_hbm.at[pl.ds(base, ROWS_PER_W)], m_vmem, sm).wait()
    lanes = lax.broadcasted_iota(jnp.int32, (16,), 0)

    bufs = (buf0, buf1)
    sin = (si0, si1)
    sout = (so0, so1)

    def chunk_rows(c):
        return pl.ds(base + c * CH, CH)

    def start_in(c):
        pltpu.make_async_copy(
            a_hbm.at[chunk_rows(c), :], bufs[c % 2], sin[c % 2]).start()

    start_in(0)
    for c in range(NCH):
        b = c % 2
        nb = (c + 1) % 2
        if c + 1 < NCH:
            if c >= 1:
                pltpu.make_async_copy(
                    a_hbm.at[chunk_rows(c - 1), :], bufs[nb], sout[nb]).wait()
            start_in(c + 1)
        pltpu.make_async_copy(
            a_hbm.at[chunk_rows(c), :], bufs[b], sin[b]).wait()

        g = c // 2
        mg = m_vmem[pl.ds(g * 16, 16)]
        half = (c % 2) * CH
        s = jnp.sum(jnp.where((lanes >= half) & (lanes < half + CH), mg, 0),
                    axis=0)

        @pl.when(s != CH)
        def _():
            @pl.loop(0, CH)
            def _(r):
                mr = jnp.sum(jnp.where(lanes == half + r, mg, 0), axis=0)

                @pl.when(mr == 0)
                def _():
                    pltpu.sync_copy(
                        x_hbm.at[pl.ds(base + c * CH + r, 1), :],
                        bufs[b].at[pl.ds(r, 1), :])

        pltpu.make_async_copy(
            bufs[b], o_hbm.at[chunk_rows(c), :], sout[b]).start()

    pltpu.make_async_copy(
        bufs[(NCH - 2) % 2],
        o_hbm.at[chunk_rows(NCH - 2), :], sout[(NCH - 2) % 2]).wait()
    pltpu.make_async_copy(
        bufs[(NCH - 1) % 2],
        o_hbm.at[chunk_rows(NCH - 1), :], sout[(NCH - 1) % 2]).wait()


def kernel(x, attack, attack_mask):
    x2 = x.reshape(SEQ, DIM)
    a2 = attack.reshape(SEQ, DIM)
    m1 = attack_mask.reshape(SEQ).astype(jnp.int32)
    mesh = plsc.VectorSubcoreMesh(core_axis_name="c", subcore_axis_name="s")
    cp = pltpu.CompilerParams()
    if "needs_layout_passes" in pltpu.CompilerParams.__dataclass_fields__:
        cp = dataclasses.replace(cp, needs_layout_passes=False)
    k = pl.kernel(
        _sc_body,
        out_type=jax.ShapeDtypeStruct((SEQ, DIM), x.dtype),
        mesh=mesh,
        scratch_types=[
            pltpu.VMEM((ROWS_PER_W,), jnp.int32),
            pltpu.VMEM((CH, DIM), jnp.float32),
            pltpu.VMEM((CH, DIM), jnp.float32),
            pltpu.SemaphoreType.DMA,
            pltpu.SemaphoreType.DMA,
            pltpu.SemaphoreType.DMA,
            pltpu.SemaphoreType.DMA,
            pltpu.SemaphoreType.DMA,
        ],
        compiler_params=cp,
    )
    out = k(m1, a2, x2)
    return out.reshape(1, SEQ, DIM)


# speedup vs baseline: 1.5108x; 1.5108x over previous
import jax
import jax.numpy as jnp
from jax.experimental import pallas as pl
from jax.experimental.pallas import tpu as pltpu

SEQ = 2048
DIM = 4096
CROWS = 128
NCH = SEQ // CROWS
NBUF = 4


def _body(m_ref, a_hbm, x_hbm, o_hbm, bufs, xbuf, sin, sout, sx):
    def start_in(c):
        pltpu.make_async_copy(
            a_hbm.at[pl.ds(c * CROWS, CROWS), :],
            bufs.at[c % NBUF], sin.at[c % NBUF]).start()

    def wait_in(c):
        pltpu.make_async_copy(
            a_hbm.at[pl.ds(c * CROWS, CROWS), :],
            bufs.at[c % NBUF], sin.at[c % NBUF]).wait()

    def start_out(c):
        pltpu.make_async_copy(
            bufs.at[c % NBUF],
            o_hbm.at[pl.ds(c * CROWS, CROWS), :], sout.at[c % NBUF]).start()

    def wait_out(c):
        pltpu.make_async_copy(
            bufs.at[c % NBUF],
            o_hbm.at[pl.ds(c * CROWS, CROWS), :], sout.at[c % NBUF]).wait()

    def process(p):
        wait_in(p)
        mc = m_ref[pl.ds(p * CROWS, CROWS), :]
        need_x = jnp.any(mc == 0)

        @pl.when(need_x)
        def _():
            cp = pltpu.make_async_copy(
                x_hbm.at[pl.ds(p * CROWS, CROWS), :], xbuf, sx)
            cp.start()
            cp.wait()
            bufs[p % NBUF] = jnp.where(mc != 0, bufs[p % NBUF], xbuf[...])

        start_out(p)

    for c in range(NCH):
        if c >= NBUF:
            wait_out(c - NBUF)
        start_in(c)
        p = c - (NBUF - 1)
        if p >= 0:
            process(p)
    for p in range(NCH - (NBUF - 1), NCH):
        process(p)
    for p in range(NCH - NBUF, NCH):
        wait_out(p)


def kernel(x, attack, attack_mask):
    x2 = x.reshape(SEQ, DIM)
    a2 = attack.reshape(SEQ, DIM)
    m2 = attack_mask.reshape(SEQ, 1).astype(jnp.int32)
    out = pl.pallas_call(
        _body,
        in_specs=[
            pl.BlockSpec(memory_space=pltpu.MemorySpace.VMEM),
            pl.BlockSpec(memory_space=pltpu.MemorySpace.HBM),
            pl.BlockSpec(memory_space=pltpu.MemorySpace.HBM),
        ],
        out_specs=pl.BlockSpec(memory_space=pltpu.MemorySpace.HBM),
        out_shape=jax.ShapeDtypeStruct((SEQ, DIM), x.dtype),
        scratch_shapes=[
            pltpu.VMEM((NBUF, CROWS, DIM), jnp.float32),
            pltpu.VMEM((CROWS, DIM), jnp.float32),
            pltpu.SemaphoreType.DMA((NBUF,)),
            pltpu.SemaphoreType.DMA((NBUF,)),
            pltpu.SemaphoreType.DMA,
        ],
    )(m2, a2, x2)
    return out.reshape(1, SEQ, DIM)


# speedup vs baseline: 1.8224x; 1.2062x over previous
import jax
import jax.numpy as jnp
from jax.experimental import pallas as pl
from jax.experimental.pallas import tpu as pltpu

SEQ = 2048
DIM = 4096
CROWS = 256
NCH = SEQ // CROWS
NBUF = 4


def _body(m_ref, a_hbm, x_hbm, o_hbm, bufs, xbuf, sin, sout, sx):
    def start_in(c):
        pltpu.make_async_copy(
            a_hbm.at[pl.ds(c * CROWS, CROWS), :],
            bufs.at[c % NBUF], sin.at[c % NBUF]).start()

    def wait_in(c):
        pltpu.make_async_copy(
            a_hbm.at[pl.ds(c * CROWS, CROWS), :],
            bufs.at[c % NBUF], sin.at[c % NBUF]).wait()

    def start_out(c):
        pltpu.make_async_copy(
            bufs.at[c % NBUF],
            o_hbm.at[pl.ds(c * CROWS, CROWS), :], sout.at[c % NBUF]).start()

    def wait_out(c):
        pltpu.make_async_copy(
            bufs.at[c % NBUF],
            o_hbm.at[pl.ds(c * CROWS, CROWS), :], sout.at[c % NBUF]).wait()

    def process(p):
        wait_in(p)
        mc = m_ref[pl.ds(p * CROWS, CROWS), :]
        need_x = jnp.any(mc == 0)

        @pl.when(need_x)
        def _():
            cp = pltpu.make_async_copy(
                x_hbm.at[pl.ds(p * CROWS, CROWS), :], xbuf, sx)
            cp.start()
            cp.wait()
            bufs[p % NBUF] = jnp.where(mc != 0, bufs[p % NBUF], xbuf[...])

        start_out(p)

    for c in range(NCH):
        if c >= NBUF:
            wait_out(c - NBUF)
        start_in(c)
        p = c - (NBUF - 1)
        if p >= 0:
            process(p)
    for p in range(NCH - (NBUF - 1), NCH):
        process(p)
    for p in range(NCH - NBUF, NCH):
        wait_out(p)


def kernel(x, attack, attack_mask):
    x2 = x.reshape(SEQ, DIM)
    a2 = attack.reshape(SEQ, DIM)
    m2 = attack_mask.reshape(SEQ, 1).astype(jnp.int32)
    out = pl.pallas_call(
        _body,
        in_specs=[
            pl.BlockSpec(memory_space=pltpu.MemorySpace.VMEM),
            pl.BlockSpec(memory_space=pltpu.MemorySpace.HBM),
            pl.BlockSpec(memory_space=pltpu.MemorySpace.HBM),
        ],
        out_specs=pl.BlockSpec(memory_space=pltpu.MemorySpace.HBM),
        out_shape=jax.ShapeDtypeStruct((SEQ, DIM), x.dtype),
        scratch_shapes=[
            pltpu.VMEM((NBUF, CROWS, DIM), jnp.float32),
            pltpu.VMEM((CROWS, DIM), jnp.float32),
            pltpu.SemaphoreType.DMA((NBUF,)),
            pltpu.SemaphoreType.DMA((NBUF,)),
            pltpu.SemaphoreType.DMA,
        ],
    )(m2, a2, x2)
    return out.reshape(1, SEQ, DIM)
